# Initial kernel scaffold; baseline (speedup 1.0000x reference)
import jax
import jax.numpy as jnp
from jax.experimental import pallas as pl


def kernel(atom_feats, bond_feats, global_feats, atom_segment_ids, bond_segment_ids):
    raise NotImplementedError("write your pallas kernel here")


# speedup vs baseline: 4.2790x; 4.2790x over previous
import functools

import jax
import jax.numpy as jnp
from jax import lax
from jax.experimental import pallas as pl
from jax.experimental.pallas import tpu as pltpu
from jax.experimental.pallas import tpu_sc as plsc

G = 1024
N = 100000
D = 128
DG = 64
CHUNK = 128
NCH = N // CHUNK
TAIL = N - NCH * CHUNK
NTILES = 16
CPT = -(-NCH // NTILES)
ROWS_PER_TILE = G // NTILES


def _sc_body(atom_hbm, bond_hbm, glob_hbm, aid_hbm, bid_hbm, out_hbm,
             acc, obuf, fbuf, ibuf, tfbuf, tibuf, gbuf):
    c = lax.axis_index("c")
    s = lax.axis_index("s")
    row0 = s * ROWS_PER_TILE

    z = jnp.zeros((16,), jnp.float32)

    def zero_row(r, carry):
        for j in range(D // 16):
            obuf[r, pl.ds(j * 16, 16)] = z
        return carry

    lax.fori_loop(0, ROWS_PER_TILE, zero_row, 0)
    pltpu.sync_copy(obuf, acc.at[pl.ds(row0, ROWS_PER_TILE)])
    plsc.subcore_barrier()

    def reduce_side(feats_hbm, ids_hbm):
        nch = jnp.minimum(NCH - s * CPT, CPT)

        def body(i, carry):
            base = (s * CPT + i) * CHUNK
            pltpu.sync_copy(feats_hbm.at[pl.ds(base, CHUNK)], fbuf)
            pltpu.sync_copy(ids_hbm.at[pl.ds(base, CHUNK)], ibuf)
            pltpu.sync_copy(fbuf, acc.at[ibuf], add=True)
            return carry

        lax.fori_loop(0, nch, body, 0)

        @pl.when(s == NTILES - 1)
        def _tail():
            pltpu.sync_copy(feats_hbm.at[pl.ds(NCH * CHUNK, TAIL)], tfbuf)
            pltpu.sync_copy(ids_hbm.at[pl.ds(NCH * CHUNK, TAIL)], tibuf)
            pltpu.sync_copy(tfbuf, acc.at[tibuf], add=True)

    @pl.when(c == 0)
    def _atoms():
        reduce_side(atom_hbm, aid_hbm)

    @pl.when(c == 1)
    def _bonds():
        reduce_side(bond_hbm, bid_hbm)

    plsc.subcore_barrier()

    pltpu.sync_copy(acc.at[pl.ds(row0, ROWS_PER_TILE)], obuf)

    @pl.when(c == 0)
    def _out_atoms():
        pltpu.sync_copy(obuf, out_hbm.at[pl.ds(row0, ROWS_PER_TILE), pl.ds(0, D)])
        pltpu.sync_copy(glob_hbm.at[pl.ds(row0, ROWS_PER_TILE)], gbuf)
        pltpu.sync_copy(gbuf, out_hbm.at[pl.ds(row0, ROWS_PER_TILE), pl.ds(2 * D, DG)])

    @pl.when(c == 1)
    def _out_bonds():
        pltpu.sync_copy(obuf, out_hbm.at[pl.ds(row0, ROWS_PER_TILE), pl.ds(D, D)])


@jax.jit
def kernel(atom_feats, bond_feats, global_feats, atom_segment_ids, bond_segment_ids):
    mesh = plsc.VectorSubcoreMesh(core_axis_name="c", subcore_axis_name="s")
    run = functools.partial(
        pl.kernel,
        out_type=jax.ShapeDtypeStruct((G, 2 * D + DG), jnp.float32),
        mesh=mesh,
        scratch_types=[
            pltpu.VMEM_SHARED((G, D), jnp.float32),
            pltpu.VMEM((ROWS_PER_TILE, D), jnp.float32),
            pltpu.VMEM((CHUNK, D), jnp.float32),
            pltpu.VMEM((CHUNK,), jnp.int32),
            pltpu.VMEM((TAIL, D), jnp.float32),
            pltpu.VMEM((TAIL,), jnp.int32),
            pltpu.VMEM((ROWS_PER_TILE, DG), jnp.float32),
        ],
    )(_sc_body)
    return run(atom_feats, bond_feats, global_feats,
               atom_segment_ids, bond_segment_ids)


# speedup vs baseline: 7.4139x; 1.7326x over previous
import functools

import jax
import jax.numpy as jnp
from jax import lax
from jax.experimental import pallas as pl
from jax.experimental.pallas import tpu as pltpu
from jax.experimental.pallas import tpu_sc as plsc

G = 1024
N = 100000
D = 128
DG = 64
CHUNK = 128
NCH = N // CHUNK
TAIL = N - NCH * CHUNK
NTILES = 16
NJ_EVEN = 48
NREM = NCH - NJ_EVEN * NTILES
ROWS_PER_TILE = G // NTILES


def _sc_body(atom_hbm, bond_hbm, glob_hbm, aid_hbm, bid_hbm, out_hbm,
             acc, obuf, fbuf0, fbuf1, ibuf0, ibuf1, tfbuf, tibuf, gbuf,
             fsem0, fsem1, isem0, isem1):
    c = lax.axis_index("c")
    s = lax.axis_index("s")
    row0 = s * ROWS_PER_TILE
    fbuf = (fbuf0, fbuf1)
    ibuf = (ibuf0, ibuf1)
    fsem = (fsem0, fsem1)
    isem = (isem0, isem1)

    z = jnp.zeros((16,), jnp.float32)

    def zero_row(r, carry):
        for j in range(D // 16):
            obuf[r, pl.ds(j * 16, 16)] = z
        return carry

    lax.fori_loop(0, ROWS_PER_TILE, zero_row, 0)
    pltpu.sync_copy(obuf, acc.at[pl.ds(row0, ROWS_PER_TILE)])
    plsc.subcore_barrier()

    def reduce_side(feats_hbm, ids_hbm):
        def base_of(j):
            return jnp.minimum((j * NTILES + s) * CHUNK, (NCH - 1) * CHUNK)

        def start(slot, j):
            b = base_of(j)
            pltpu.async_copy(feats_hbm.at[pl.ds(b, CHUNK)], fbuf[slot], fsem[slot])
            pltpu.async_copy(ids_hbm.at[pl.ds(b, CHUNK)], ibuf[slot], isem[slot])

        def wait(slot, j):
            b = base_of(j)
            pltpu.make_async_copy(feats_hbm.at[pl.ds(b, CHUNK)], fbuf[slot], fsem[slot]).wait()
            pltpu.make_async_copy(ids_hbm.at[pl.ds(b, CHUNK)], ibuf[slot], isem[slot]).wait()

        def scatter(slot):
            pltpu.sync_copy(fbuf[slot], acc.at[ibuf[slot]], add=True)

        start(0, 0)

        def body(i, carry):
            start(1, 2 * i + 1)
            wait(0, 2 * i)
            scatter(0)
            start(0, 2 * i + 2)
            wait(1, 2 * i + 1)
            scatter(1)
            return carry

        lax.fori_loop(0, NJ_EVEN // 2, body, 0)
        wait(0, NJ_EVEN)

        @pl.when(s < NREM)
        def _odd():
            scatter(0)

        @pl.when(s == NTILES - 1)
        def _tail():
            pltpu.sync_copy(feats_hbm.at[pl.ds(NCH * CHUNK, TAIL)], tfbuf)
            pltpu.sync_copy(ids_hbm.at[pl.ds(NCH * CHUNK, TAIL)], tibuf)
            pltpu.sync_copy(tfbuf, acc.at[tibuf], add=True)

    @pl.when(c == 0)
    def _atoms():
        reduce_side(atom_hbm, aid_hbm)

    @pl.when(c == 1)
    def _bonds():
        reduce_side(bond_hbm, bid_hbm)

    plsc.subcore_barrier()

    pltpu.sync_copy(acc.at[pl.ds(row0, ROWS_PER_TILE)], obuf)

    @pl.when(c == 0)
    def _out_atoms():
        pltpu.sync_copy(obuf, out_hbm.at[pl.ds(row0, ROWS_PER_TILE), pl.ds(0, D)])
        pltpu.sync_copy(glob_hbm.at[pl.ds(row0, ROWS_PER_TILE)], gbuf)
        pltpu.sync_copy(gbuf, out_hbm.at[pl.ds(row0, ROWS_PER_TILE), pl.ds(2 * D, DG)])

    @pl.when(c == 1)
    def _out_bonds():
        pltpu.sync_copy(obuf, out_hbm.at[pl.ds(row0, ROWS_PER_TILE), pl.ds(D, D)])


@jax.jit
def kernel(atom_feats, bond_feats, global_feats, atom_segment_ids, bond_segment_ids):
    mesh = plsc.VectorSubcoreMesh(core_axis_name="c", subcore_axis_name="s")
    run = functools.partial(
        pl.kernel,
        out_type=jax.ShapeDtypeStruct((G, 2 * D + DG), jnp.float32),
        mesh=mesh,
        scratch_types=[
            pltpu.VMEM_SHARED((G, D), jnp.float32),
            pltpu.VMEM((ROWS_PER_TILE, D), jnp.float32),
            pltpu.VMEM((CHUNK, D), jnp.float32),
            pltpu.VMEM((CHUNK, D), jnp.float32),
            pltpu.VMEM((CHUNK,), jnp.int32),
            pltpu.VMEM((CHUNK,), jnp.int32),
            pltpu.VMEM((TAIL, D), jnp.float32),
            pltpu.VMEM((TAIL,), jnp.int32),
            pltpu.VMEM((ROWS_PER_TILE, DG), jnp.float32),
            pltpu.SemaphoreType.DMA,
            pltpu.SemaphoreType.DMA,
            pltpu.SemaphoreType.DMA,
            pltpu.SemaphoreType.DMA,
        ],
    )(_sc_body)
    return run(atom_feats, bond_feats, global_feats,
               atom_segment_ids, bond_segment_ids)
